# baseline (speedup 1.0000x reference)
import functools

import jax
import jax.numpy as jnp
from jax import lax
from jax.experimental import pallas as pl
from jax.experimental.pallas import tpu as pltpu
from jax.experimental.pallas import tpu_sc as plsc


def _mlp_body(table_ref, w1_ref, b1_ref, w2_ref, b2_ref, out_ref):
    h = jnp.dot(table_ref[...], w1_ref[...], preferred_element_type=jnp.float32)
    h = h + b1_ref[...]
    h = h * jax.nn.sigmoid(h)
    o = jnp.dot(h, w2_ref[...], preferred_element_type=jnp.float32)
    o = o + b2_ref[...]
    out_ref[...] = o * jax.nn.sigmoid(o)


def _mlp_on_table(table, W1, b1, W2, b2):
    n, d = table.shape
    return pl.pallas_call(
        _mlp_body,
        out_shape=jax.ShapeDtypeStruct((n, W2.shape[1]), jnp.float32),
    )(table, W1, b1.reshape(1, -1), W2, b2.reshape(1, -1))


def _make_gather(V, D, B):
    info = plsc.get_sparse_core_info()
    NC, NS = info.num_cores, info.num_subcores
    NW = NC * NS
    assert B % (8 * NW) == 0
    b_per_w = B // NW
    C = 128
    NCH = b_per_w // C
    mesh = plsc.VectorSubcoreMesh(core_axis_name="c", subcore_axis_name="s")

    @functools.partial(
        pl.kernel,
        mesh=mesh,
        out_type=jax.ShapeDtypeStruct((B, D), jnp.float32),
        scratch_types=[
            pltpu.VMEM((b_per_w,), jnp.int32),
            pltpu.VMEM((2, C, D), jnp.float32),
            pltpu.SemaphoreType.DMA,
            pltpu.SemaphoreType.DMA,
            pltpu.SemaphoreType.DMA,
            pltpu.SemaphoreType.DMA,
        ],
    )
    def gather(table_hbm, idx_hbm, out_hbm, idx_v, rows_v, g0, g1, w0, w1):
        wid = lax.axis_index("s") * NC + lax.axis_index("c")
        base = wid * b_per_w
        gsem = (g0, g1)
        wsem = (w0, w1)
        pltpu.sync_copy(idx_hbm.at[pl.ds(base, b_per_w)], idx_v)

        def start_gather(j):
            return pltpu.async_copy(
                table_hbm.at[idx_v.at[pl.ds(j * C, C)]],
                rows_v.at[j % 2],
                gsem[j % 2],
            )

        def start_write(j):
            return pltpu.async_copy(
                rows_v.at[j % 2],
                out_hbm.at[pl.ds(base + j * C, C)],
                wsem[j % 2],
            )

        g = [None] * NCH
        w = [None] * NCH
        g[0] = start_gather(0)
        for j in range(NCH):
            if j + 1 < NCH:
                if j >= 1:
                    w[j - 1].wait()
                g[j + 1] = start_gather(j + 1)
            g[j].wait()
            w[j] = start_write(j)
        w[NCH - 2].wait()
        w[NCH - 1].wait()

    return gather


def kernel(table, W1, b1, W2, b2, diffusion_step):
    t2 = _mlp_on_table(table, W1, b1, W2, b2)
    B = diffusion_step.shape[0]
    V, D = t2.shape
    idx = diffusion_step.astype(jnp.int32)
    return _make_gather(V, D, B)(t2, idx)


# speedup vs baseline: 1.1541x; 1.1541x over previous
import functools

import jax
import jax.numpy as jnp
from jax import lax
from jax.experimental import pallas as pl
from jax.experimental.pallas import tpu as pltpu
from jax.experimental.pallas import tpu_sc as plsc


def _mlp_body(table_ref, w1_ref, b1_ref, w2_ref, b2_ref, out_ref):
    h = jnp.dot(table_ref[...], w1_ref[...], preferred_element_type=jnp.float32)
    h = h + b1_ref[...]
    h = h * jax.nn.sigmoid(h)
    o = jnp.dot(h, w2_ref[...], preferred_element_type=jnp.float32)
    o = o + b2_ref[...]
    out_ref[...] = o * jax.nn.sigmoid(o)


def _mlp_on_table(table, W1, b1, W2, b2):
    n, d = table.shape
    return pl.pallas_call(
        _mlp_body,
        out_shape=jax.ShapeDtypeStruct((n, W2.shape[1]), jnp.float32),
    )(table, W1, b1.reshape(1, -1), W2, b2.reshape(1, -1))


def _make_gather(V, D, B):
    info = plsc.get_sparse_core_info()
    NC, NS = info.num_cores, info.num_subcores
    NW = NC * NS
    assert B % (8 * NW) == 0
    b_per_w = B // NW
    C = 128
    NCH = b_per_w // C
    mesh = plsc.VectorSubcoreMesh(core_axis_name="c", subcore_axis_name="s")

    @functools.partial(
        pl.kernel,
        mesh=mesh,
        out_type=jax.ShapeDtypeStruct((B, D), jnp.float32),
        scratch_types=[
            pltpu.VMEM((b_per_w,), jnp.int32),
            pltpu.VMEM((2, C, D), jnp.float32),
            pltpu.VMEM_SHARED((V, D), jnp.float32),
            pltpu.SemaphoreType.DMA,
            pltpu.SemaphoreType.DMA,
            pltpu.SemaphoreType.DMA,
            pltpu.SemaphoreType.DMA,
        ],
    )
    def gather(table_hbm, idx_hbm, out_hbm, idx_v, rows_v, table_sp, g0, g1, w0, w1):
        sid = lax.axis_index("s")
        wid = sid * NC + lax.axis_index("c")
        base = wid * b_per_w
        gsem = (g0, g1)
        wsem = (w0, w1)
        @pl.when(sid == 0)
        def _load_table():
            pltpu.sync_copy(table_hbm, table_sp)

        pltpu.sync_copy(idx_hbm.at[pl.ds(base, b_per_w)], idx_v)
        plsc.subcore_barrier()

        def start_gather(j):
            return pltpu.async_copy(
                table_sp.at[idx_v.at[pl.ds(j * C, C)]],
                rows_v.at[j % 2],
                gsem[j % 2],
            )

        def start_write(j):
            return pltpu.async_copy(
                rows_v.at[j % 2],
                out_hbm.at[pl.ds(base + j * C, C)],
                wsem[j % 2],
            )

        g = [None] * NCH
        w = [None] * NCH
        g[0] = start_gather(0)
        for j in range(NCH):
            if j + 1 < NCH:
                if j >= 1:
                    w[j - 1].wait()
                g[j + 1] = start_gather(j + 1)
            g[j].wait()
            w[j] = start_write(j)
        w[NCH - 2].wait()
        w[NCH - 1].wait()

    return gather


def kernel(table, W1, b1, W2, b2, diffusion_step):
    t2 = _mlp_on_table(table, W1, b1, W2, b2)
    B = diffusion_step.shape[0]
    V, D = t2.shape
    idx = diffusion_step.astype(jnp.int32)
    return _make_gather(V, D, B)(t2, idx)


# speedup vs baseline: 1.1604x; 1.0055x over previous
import functools

import jax
import jax.numpy as jnp
from jax import lax
from jax.experimental import pallas as pl
from jax.experimental.pallas import tpu as pltpu
from jax.experimental.pallas import tpu_sc as plsc


def _mlp_on_table(table, W1, b1, W2, b2, n_pad):
    n, d = table.shape

    def body(table_ref, w1_ref, b1_ref, w2_ref, b2_ref, out_ref):
        h = jnp.dot(table_ref[...], w1_ref[...], preferred_element_type=jnp.float32)
        h = h + b1_ref[...]
        h = h * jax.nn.sigmoid(h)
        o = jnp.dot(h, w2_ref[...], preferred_element_type=jnp.float32)
        o = o + b2_ref[...]
        out_ref[0:n, :] = o * jax.nn.sigmoid(o)

    return pl.pallas_call(
        body,
        out_shape=jax.ShapeDtypeStruct((n_pad, W2.shape[1]), jnp.float32),
    )(table, W1, b1.reshape(1, -1), W2, b2.reshape(1, -1))


def _make_gather(V, D, B):
    info = plsc.get_sparse_core_info()
    NC, NS = info.num_cores, info.num_subcores
    NW = NC * NS
    assert B % (8 * NW) == 0
    assert V % NS == 0
    v_per_s = V // NS
    b_per_w = B // NW
    C = 64
    NCH = b_per_w // C
    mesh = plsc.VectorSubcoreMesh(core_axis_name="c", subcore_axis_name="s")

    @functools.partial(
        pl.kernel,
        mesh=mesh,
        out_type=jax.ShapeDtypeStruct((B, D), jnp.float32),
        scratch_types=[
            pltpu.VMEM((b_per_w,), jnp.int32),
            pltpu.VMEM((2, C, D), jnp.float32),
            pltpu.VMEM_SHARED((V, D), jnp.float32),
            pltpu.SemaphoreType.DMA,
            pltpu.SemaphoreType.DMA,
            pltpu.SemaphoreType.DMA,
            pltpu.SemaphoreType.DMA,
        ],
    )
    def gather(table_hbm, idx_hbm, out_hbm, idx_v, rows_v, table_sp, g0, g1, w0, w1):
        sid = lax.axis_index("s")
        wid = sid * NC + lax.axis_index("c")
        base = wid * b_per_w
        gsem = (g0, g1)
        wsem = (w0, w1)
        pltpu.sync_copy(
            table_hbm.at[pl.ds(sid * v_per_s, v_per_s)],
            table_sp.at[pl.ds(sid * v_per_s, v_per_s)],
        )
        pltpu.sync_copy(idx_hbm.at[pl.ds(base, b_per_w)], idx_v)
        plsc.subcore_barrier()

        def start_gather(j):
            return pltpu.async_copy(
                table_sp.at[idx_v.at[pl.ds(j * C, C)]],
                rows_v.at[j % 2],
                gsem[j % 2],
            )

        def start_write(j):
            return pltpu.async_copy(
                rows_v.at[j % 2],
                out_hbm.at[pl.ds(base + j * C, C)],
                wsem[j % 2],
            )

        g = [None] * NCH
        w = [None] * NCH
        g[0] = start_gather(0)
        for j in range(NCH):
            if j + 1 < NCH:
                if j >= 1:
                    w[j - 1].wait()
                g[j + 1] = start_gather(j + 1)
            g[j].wait()
            w[j] = start_write(j)
        w[NCH - 2].wait()
        w[NCH - 1].wait()

    return gather


def kernel(table, W1, b1, W2, b2, diffusion_step):
    n = table.shape[0]
    n_pad = (n + 127) // 128 * 128
    t2 = _mlp_on_table(table, W1, b1, W2, b2, n_pad)
    B = diffusion_step.shape[0]
    V, D = t2.shape
    idx = diffusion_step.astype(jnp.int32)
    return _make_gather(V, D, B)(t2, idx)
